# baseline (speedup 1.0000x reference)
import jax
import jax.numpy as jnp
from jax.experimental import pallas as pl
from jax.experimental.pallas import tpu as pltpu

_N = 16777216
_ROWS = 4096
_COLS = 4096
_BLOCK_ROWS = 128
_GRID = _ROWS // _BLOCK_ROWS
_NB = 17


def _tc_body(b_ref, v_ref, out_ref):
    i = pl.program_id(0)
    v = v_ref[...]
    idx1 = jnp.zeros(v.shape, jnp.int32)
    idx2 = jnp.zeros(v.shape, jnp.int32)
    for j in range(_NB):
        idx1 = idx1 + (b_ref[j] < v).astype(jnp.int32)
    for j in reversed(range(_NB)):
        idx2 = idx2 + (v <= b_ref[j]).astype(jnp.int32)
    idx2 = _NB - idx2
    ok = jnp.min(jnp.where(idx1 == idx2, 1, 0)).astype(jnp.int32)

    @pl.when(i == 0)
    def _():
        out_ref[0, 0] = 1

    out_ref[0, 0] = out_ref[0, 0] & ok


def kernel(vals, boundaries):
    v2 = vals.reshape(_ROWS, _COLS)
    out = pl.pallas_call(
        _tc_body,
        grid=(_GRID,),
        in_specs=[
            pl.BlockSpec(memory_space=pltpu.SMEM),
            pl.BlockSpec((_BLOCK_ROWS, _COLS), lambda i: (i, 0)),
        ],
        out_specs=pl.BlockSpec(memory_space=pltpu.SMEM),
        out_shape=jax.ShapeDtypeStruct((1, 1), jnp.int32),
    )(boundaries, v2)
    return out.reshape(()).astype(jnp.bool_)
